# Initial kernel scaffold; baseline (speedup 1.0000x reference)
import jax
import jax.numpy as jnp
from jax.experimental import pallas as pl


def kernel(pred, target):
    raise NotImplementedError("write your pallas kernel here")


# speedup vs baseline: 1.4369x; 1.4369x over previous
import functools

import jax
import jax.numpy as jnp
from jax.experimental import pallas as pl


TILE_I = 512


def _chamfer_body(pred_ref, tgt_ref, minp_ref, mint_ref):
    it = pl.program_id(1)
    px = pred_ref[0, :, 0:1]
    py = pred_ref[0, :, 1:2]
    pz = pred_ref[0, :, 2:3]
    tx = tgt_ref[0, 0:1, :]
    ty = tgt_ref[0, 1:2, :]
    tz = tgt_ref[0, 2:3, :]
    d = (px - tx) ** 2 + (py - ty) ** 2 + (pz - tz) ** 2
    minp_ref[0, 0, :] = jnp.min(d, axis=1)
    colmin = jnp.min(d, axis=0, keepdims=True)

    @pl.when(it == 0)
    def _init():
        mint_ref[0] = colmin

    @pl.when(it != 0)
    def _acc():
        mint_ref[0] = jnp.minimum(mint_ref[0], colmin)


@functools.partial(jax.jit, static_argnames=("interpret",))
def kernel(pred, target, interpret=False):
    B, N, _ = pred.shape
    M = target.shape[1]
    tgt_t = jnp.swapaxes(target, 1, 2)
    grid = (B, N // TILE_I)
    minp, mint = pl.pallas_call(
        _chamfer_body,
        grid=grid,
        in_specs=[
            pl.BlockSpec((1, TILE_I, 3), lambda b, it: (b, it, 0)),
            pl.BlockSpec((1, 3, M), lambda b, it: (b, 0, 0)),
        ],
        out_specs=[
            pl.BlockSpec((1, 1, TILE_I),
                         lambda b, it: (b * (N // TILE_I) + it, 0, 0)),
            pl.BlockSpec((1, 1, M), lambda b, it: (b, 0, 0)),
        ],
        out_shape=[
            jax.ShapeDtypeStruct((B * (N // TILE_I), 1, TILE_I), jnp.float32),
            jax.ShapeDtypeStruct((B, 1, M), jnp.float32),
        ],
        interpret=interpret,
    )(pred, tgt_t)
    return jnp.mean(minp) + jnp.mean(mint)


# speedup vs baseline: 1.6130x; 1.1226x over previous
import functools

import jax
import jax.numpy as jnp
from jax.experimental import pallas as pl


TILE_I = 512


def _chamfer_body(pred_ref, mt_ref, tn_ref, minp_ref, mint_ref):
    it = pl.program_id(1)
    p = pred_ref[0]
    e = jnp.dot(p, mt_ref[0], preferred_element_type=jnp.float32)
    px = pred_ref[0, :, 0:1]
    py = pred_ref[0, :, 1:2]
    pz = pred_ref[0, :, 2:3]
    pn_col = px * px + py * py + pz * pz
    g = (e + tn_ref[0]) + pn_col
    minp_ref[0, 0, :] = jnp.min(g, axis=1)
    colmin = jnp.min(g, axis=0, keepdims=True)

    @pl.when(it == 0)
    def _init():
        mint_ref[0] = colmin

    @pl.when(it != 0)
    def _acc():
        mint_ref[0] = jnp.minimum(mint_ref[0], colmin)


@functools.partial(jax.jit, static_argnames=("interpret",))
def kernel(pred, target, interpret=False):
    B, N, _ = pred.shape
    M = target.shape[1]
    pred8 = jnp.pad(pred, ((0, 0), (0, 0), (0, 5)))
    mt = -2.0 * jnp.swapaxes(target, 1, 2)
    mt8 = jnp.pad(mt, ((0, 0), (0, 5), (0, 0)))
    tn = jnp.sum(target * target, axis=2)[:, None, :]
    grid = (B, N // TILE_I)
    minp, mint = pl.pallas_call(
        _chamfer_body,
        grid=grid,
        in_specs=[
            pl.BlockSpec((1, TILE_I, 8), lambda b, it: (b, it, 0)),
            pl.BlockSpec((1, 8, M), lambda b, it: (b, 0, 0)),
            pl.BlockSpec((1, 1, M), lambda b, it: (b, 0, 0)),
        ],
        out_specs=[
            pl.BlockSpec((1, 1, TILE_I),
                         lambda b, it: (b * (N // TILE_I) + it, 0, 0)),
            pl.BlockSpec((1, 1, M), lambda b, it: (b, 0, 0)),
        ],
        out_shape=[
            jax.ShapeDtypeStruct((B * (N // TILE_I), 1, TILE_I), jnp.float32),
            jax.ShapeDtypeStruct((B, 1, M), jnp.float32),
        ],
        interpret=interpret,
    )(pred8, mt8, tn)
    return jnp.mean(minp) + jnp.mean(mint)
